# baseline (speedup 1.0000x reference)
import functools

import jax
import jax.numpy as jnp
from jax import lax
from jax.experimental import pallas as pl
from jax.experimental.pallas import tpu as pltpu
from jax.experimental.pallas import tpu_sc as plsc

_EMB = 64
_LANES = 16
_SEGS = _EMB // _LANES


def kernel(center_words, context_words, in_emb, out_emb):
    B = center_words.shape[0]
    NC, NS = 2, 16
    NW = NC * NS
    b_per_w = B // NW
    CHUNK = 128
    n_chunks = b_per_w // CHUNK

    mesh = plsc.VectorSubcoreMesh(core_axis_name="c", subcore_axis_name="s")

    @functools.partial(
        pl.kernel,
        mesh=mesh,
        compiler_params=pltpu.CompilerParams(needs_layout_passes=False,
                                             use_tc_tiling_on_sc=False),
        out_type=jax.ShapeDtypeStruct((B,), jnp.float32),
        scratch_types=[
            pltpu.VMEM((n_chunks, CHUNK), jnp.int32),
            pltpu.VMEM((n_chunks, CHUNK), jnp.int32),
            pltpu.VMEM((b_per_w, _EMB), jnp.float32),
            pltpu.VMEM((b_per_w, _EMB), jnp.float32),
            pltpu.VMEM((b_per_w,), jnp.float32),
            pltpu.SemaphoreType.DMA,
        ],
    )
    def sc_kernel(center_hbm, context_hbm, in_hbm, out_hbm, scores_hbm,
                  cidx_v, xidx_v, crows_v, xrows_v, sv, sem):
        wid = lax.axis_index("s") * NC + lax.axis_index("c")
        base = wid * b_per_w

        for j in range(n_chunks):
            pltpu.sync_copy(center_hbm.at[pl.ds(base + j * CHUNK, CHUNK)],
                            cidx_v.at[j])
            pltpu.sync_copy(context_hbm.at[pl.ds(base + j * CHUNK, CHUNK)],
                            xidx_v.at[j])

        copies = []
        for j in range(n_chunks):
            copies.append(pltpu.async_copy(
                in_hbm.at[cidx_v.at[j]],
                crows_v.at[pl.ds(j * CHUNK, CHUNK)], sem))
            copies.append(pltpu.async_copy(
                out_hbm.at[xidx_v.at[j]],
                xrows_v.at[pl.ds(j * CHUNK, CHUNK)], sem))
        for c in copies:
            c.wait()

        lane = lax.iota(jnp.int32, _LANES)

        def group(g, carry):
            base_r = g * _LANES
            res = jnp.zeros((_LANES,), jnp.float32)
            for i in range(_LANES):
                r = base_r + i
                acc = (crows_v[r, pl.ds(0, _LANES)]
                       * xrows_v[r, pl.ds(0, _LANES)])
                for s in range(1, _SEGS):
                    acc = acc + (crows_v[r, pl.ds(s * _LANES, _LANES)]
                                 * xrows_v[r, pl.ds(s * _LANES, _LANES)])
                res = jnp.where(lane == i, jnp.sum(acc), res)
            sv[pl.ds(base_r, _LANES)] = res
            return carry

        lax.fori_loop(0, b_per_w // _LANES, group, 0)

        pltpu.sync_copy(sv, scores_hbm.at[pl.ds(base, b_per_w)])

    return sc_kernel(center_words, context_words, in_emb, out_emb)


# speedup vs baseline: 1.5837x; 1.5837x over previous
import functools

import jax
import jax.numpy as jnp
from jax import lax
from jax.experimental import pallas as pl
from jax.experimental.pallas import tpu as pltpu
from jax.experimental.pallas import tpu_sc as plsc

_EMB = 64
_LANES = 16
_SEGS = _EMB // _LANES
_CH = 32


def kernel(center_words, context_words, in_emb, out_emb):
    B = center_words.shape[0]
    NC, NS = 2, 16
    NW = NC * NS
    b_per_w = B // NW
    n_chunks = b_per_w // _CH

    mesh = plsc.VectorSubcoreMesh(core_axis_name="c", subcore_axis_name="s")

    @functools.partial(
        pl.kernel,
        mesh=mesh,
        compiler_params=pltpu.CompilerParams(needs_layout_passes=False),
        out_type=jax.ShapeDtypeStruct((B,), jnp.float32),
        scratch_types=[
            pltpu.VMEM((b_per_w,), jnp.int32),
            pltpu.VMEM((b_per_w,), jnp.int32),
            pltpu.VMEM((2, _CH, _EMB), jnp.float32),
            pltpu.VMEM((2, _CH, _EMB), jnp.float32),
            pltpu.VMEM((b_per_w,), jnp.float32),
            pltpu.SemaphoreType.DMA,
            pltpu.SemaphoreType.DMA,
        ],
    )
    def sc_kernel(center_hbm, context_hbm, in_hbm, out_hbm, scores_hbm,
                  cidx_v, xidx_v, cbuf, xbuf, sv, sem0, sem1):
        wid = lax.axis_index("s") * NC + lax.axis_index("c")
        base = wid * b_per_w

        pltpu.sync_copy(center_hbm.at[pl.ds(base, b_per_w)], cidx_v)
        pltpu.sync_copy(context_hbm.at[pl.ds(base, b_per_w)], xidx_v)

        sems = (sem0, sem1)
        lane = lax.iota(jnp.int32, _LANES)

        def issue(c, slot):
            for g in range(_CH // _LANES):
                civ = cidx_v[pl.ds(c * _CH + g * _LANES, _LANES)]
                xiv = xidx_v[pl.ds(c * _CH + g * _LANES, _LANES)]
                for i in range(_LANES):
                    li = g * _LANES + i
                    pltpu.async_copy(in_hbm.at[civ[i]],
                                     cbuf.at[slot, li], sems[slot])
                    pltpu.async_copy(out_hbm.at[xiv[i]],
                                     xbuf.at[slot, li], sems[slot])

        def drain(slot):
            pltpu.make_async_copy(in_hbm.at[pl.ds(0, _CH)], cbuf.at[slot],
                                  sems[slot]).wait()
            pltpu.make_async_copy(out_hbm.at[pl.ds(0, _CH)], xbuf.at[slot],
                                  sems[slot]).wait()

        def compute(c, slot):
            for g in range(_CH // _LANES):
                res = jnp.zeros((_LANES,), jnp.float32)
                for i in range(_LANES):
                    li = g * _LANES + i
                    acc = (cbuf[slot, li, pl.ds(0, _LANES)]
                           * xbuf[slot, li, pl.ds(0, _LANES)])
                    for s in range(1, _SEGS):
                        acc = acc + (
                            cbuf[slot, li, pl.ds(s * _LANES, _LANES)]
                            * xbuf[slot, li, pl.ds(s * _LANES, _LANES)])
                    res = jnp.where(lane == i, jnp.sum(acc), res)
                sv[pl.ds(c * _CH + g * _LANES, _LANES)] = res

        issue(0, 0)
        issue(1, 1)

        def step(t, carry):
            drain(0)
            compute(2 * t, 0)

            @pl.when(t < n_chunks // 2 - 1)
            def _():
                issue(2 * t + 2, 0)

            drain(1)
            compute(2 * t + 1, 1)

            @pl.when(t < n_chunks // 2 - 1)
            def _():
                issue(2 * t + 3, 1)

            return carry

        lax.fori_loop(0, n_chunks // 2, step, 0)

        pltpu.sync_copy(sv, scores_hbm.at[pl.ds(base, b_per_w)])

    return sc_kernel(center_words, context_words, in_emb, out_emb)
